# baseline (speedup 1.0000x reference)
import functools

import jax
import jax.numpy as jnp
from jax import lax
from jax.experimental import pallas as pl
from jax.experimental.pallas import tpu as pltpu
from jax.experimental.pallas import tpu_sc as plsc

_N = 1600000
_NC = 2
_NS = 16
_NW = _NC * _NS
_PER_W = _N // _NW
_C = 10000
_NCHUNK = _PER_W // _C


def _sc_body(p_hbm, a_hbm, b_hbm, o_hbm, p_v, a_v, b_v, o_v, in_sem, out_sem):
    wid = lax.axis_index("s") * _NC + lax.axis_index("c")
    wbase = wid * _PER_W

    def in_copies(g, slot):
        hsl = pl.ds(wbase + g * _C, _C)
        vsl = pl.ds(slot * _C, _C)
        return [
            pltpu.make_async_copy(p_hbm.at[hsl], p_v.at[vsl], in_sem.at[slot]),
            pltpu.make_async_copy(a_hbm.at[hsl], a_v.at[vsl], in_sem.at[slot]),
            pltpu.make_async_copy(b_hbm.at[hsl], b_v.at[vsl], in_sem.at[slot]),
        ]

    def out_copy(g, slot):
        return pltpu.make_async_copy(
            o_v.at[pl.ds(slot * _C, _C)],
            o_hbm.at[pl.ds(wbase + g * _C, _C)], out_sem.at[slot])

    def compute(soff):
        def inner(i, _):
            for j in range(5):
                s = pl.ds(soff + i * 80 + j * 16, 16)
                o_v[s] = a_v[s] + b_v[s] * jnp.maximum(p_v[s], 0.0)
            return 0
        lax.fori_loop(0, _C // 80, inner, 0, unroll=False)

    for cp in in_copies(0, 0):
        cp.start()

    def chunk_step(g, _):
        slot = lax.rem(g, 2)

        @pl.when(g + 1 < _NCHUNK)
        def _():
            for cp in in_copies(g + 1, 1 - slot):
                cp.start()

        for cp in in_copies(g, slot):
            cp.wait()

        @pl.when(g >= 2)
        def _():
            out_copy(g - 2, slot).wait()

        compute(slot * _C)
        out_copy(g, slot).start()
        return 0

    lax.fori_loop(0, _NCHUNK, chunk_step, 0, unroll=False)
    out_copy(_NCHUNK - 2, (_NCHUNK - 2) % 2).wait()
    out_copy(_NCHUNK - 1, (_NCHUNK - 1) % 2).wait()


def kernel(products, A, B):
    mesh = plsc.VectorSubcoreMesh(core_axis_name="c", subcore_axis_name="s")
    run = functools.partial(
        pl.kernel,
        mesh=mesh,
        out_type=jax.ShapeDtypeStruct((_N,), jnp.float32),
        scratch_types=[
            pltpu.VMEM((2 * _C,), jnp.float32),
            pltpu.VMEM((2 * _C,), jnp.float32),
            pltpu.VMEM((2 * _C,), jnp.float32),
            pltpu.VMEM((2 * _C,), jnp.float32),
            pltpu.SemaphoreType.DMA((2,)),
            pltpu.SemaphoreType.DMA((2,)),
        ],
    )(_sc_body)
    return run(products, A, B)


# speedup vs baseline: 1.4212x; 1.4212x over previous
import functools

import jax
import jax.numpy as jnp
from jax import lax
from jax.experimental import pallas as pl
from jax.experimental.pallas import tpu as pltpu
from jax.experimental.pallas import tpu_sc as plsc

_N = 1600000
_NC = 2
_NS = 16
_NW = _NC * _NS
_PER_W = _N // _NW
_C = 10000
_NCHUNK = _PER_W // _C


def _sc_body(p_hbm, a_hbm, b_hbm, o_hbm, p_v, a_v, b_v, o_v, in_sem, out_sem):
    wid = lax.axis_index("s") * _NC + lax.axis_index("c")
    wbase = wid * _PER_W

    def in_copies(g, slot):
        hsl = pl.ds(wbase + g * _C, _C)
        vsl = pl.ds(slot * _C, _C)
        return [
            pltpu.make_async_copy(p_hbm.at[hsl], p_v.at[vsl], in_sem.at[slot]),
            pltpu.make_async_copy(a_hbm.at[hsl], a_v.at[vsl], in_sem.at[slot]),
            pltpu.make_async_copy(b_hbm.at[hsl], b_v.at[vsl], in_sem.at[slot]),
        ]

    def out_copy(g, slot):
        return pltpu.make_async_copy(
            o_v.at[pl.ds(slot * _C, _C)],
            o_hbm.at[pl.ds(wbase + g * _C, _C)], out_sem.at[slot])

    def compute(soff):
        def inner(i, _):
            for j in range(5):
                s = pl.ds(pl.multiple_of(soff + i * 80 + j * 16, 16), 16)
                o_v[s] = a_v[s] + b_v[s] * jnp.maximum(p_v[s], 0.0)
            return 0
        lax.fori_loop(0, _C // 80, inner, 0, unroll=False)

    for cp in in_copies(0, 0):
        cp.start()

    def chunk_step(g, _):
        slot = lax.rem(g, 2)

        @pl.when(g + 1 < _NCHUNK)
        def _():
            for cp in in_copies(g + 1, 1 - slot):
                cp.start()

        for cp in in_copies(g, slot):
            cp.wait()

        @pl.when(g >= 2)
        def _():
            out_copy(g - 2, slot).wait()

        @pl.when(slot == 0)
        def _():
            compute(0)

        @pl.when(slot == 1)
        def _():
            compute(_C)
        out_copy(g, slot).start()
        return 0

    lax.fori_loop(0, _NCHUNK, chunk_step, 0, unroll=False)
    out_copy(_NCHUNK - 2, (_NCHUNK - 2) % 2).wait()
    out_copy(_NCHUNK - 1, (_NCHUNK - 1) % 2).wait()


def kernel(products, A, B):
    mesh = plsc.VectorSubcoreMesh(core_axis_name="c", subcore_axis_name="s")
    run = functools.partial(
        pl.kernel,
        mesh=mesh,
        out_type=jax.ShapeDtypeStruct((_N,), jnp.float32),
        scratch_types=[
            pltpu.VMEM((2 * _C,), jnp.float32),
            pltpu.VMEM((2 * _C,), jnp.float32),
            pltpu.VMEM((2 * _C,), jnp.float32),
            pltpu.VMEM((2 * _C,), jnp.float32),
            pltpu.SemaphoreType.DMA((2,)),
            pltpu.SemaphoreType.DMA((2,)),
        ],
    )(_sc_body)
    return run(products, A, B)


# speedup vs baseline: 4.2172x; 2.9672x over previous
import jax
import jax.numpy as jnp
from jax.experimental import pallas as pl


_COLS = 128
_BLOCK_ROWS = 1792


def _body(p_ref, a_ref, b_ref, o_ref):
    o_ref[...] = a_ref[...] + b_ref[...] * jnp.maximum(p_ref[...], 0.0)


def kernel(products, A, B):
    n = products.shape[0]
    rows = n // _COLS
    p2 = products.reshape(rows, _COLS)
    a2 = A.reshape(rows, _COLS)
    b2 = B.reshape(rows, _COLS)
    grid = (rows + _BLOCK_ROWS - 1) // _BLOCK_ROWS
    spec = pl.BlockSpec((_BLOCK_ROWS, _COLS), lambda i: (i, 0))
    out = pl.pallas_call(
        _body,
        grid=(grid,),
        in_specs=[spec, spec, spec],
        out_specs=spec,
        out_shape=jax.ShapeDtypeStruct((rows, _COLS), jnp.float32),
    )(p2, a2, b2)
    return out.reshape(n)


# speedup vs baseline: 4.5084x; 1.0691x over previous
import jax
import jax.numpy as jnp
from jax.experimental import pallas as pl


_COLS = 128
_BLOCK_ROWS = 2560


def _body(p_ref, a_ref, b_ref, o_ref):
    o_ref[...] = a_ref[...] + b_ref[...] * jnp.maximum(p_ref[...], 0.0)


def kernel(products, A, B):
    n = products.shape[0]
    rows = n // _COLS
    p2 = products.reshape(rows, _COLS)
    a2 = A.reshape(rows, _COLS)
    b2 = B.reshape(rows, _COLS)
    grid = (rows + _BLOCK_ROWS - 1) // _BLOCK_ROWS
    spec = pl.BlockSpec((_BLOCK_ROWS, _COLS), lambda i: (i, 0))
    out = pl.pallas_call(
        _body,
        grid=(grid,),
        in_specs=[spec, spec, spec],
        out_specs=spec,
        out_shape=jax.ShapeDtypeStruct((rows, _COLS), jnp.float32),
    )(p2, a2, b2)
    return out.reshape(n)


# speedup vs baseline: 4.6381x; 1.0288x over previous
import jax
import jax.numpy as jnp
from jax.experimental import pallas as pl


_COLS = 128
_BLOCK_ROWS = 3200


def _body(p_ref, a_ref, b_ref, o_ref):
    o_ref[...] = a_ref[...] + b_ref[...] * jnp.maximum(p_ref[...], 0.0)


def kernel(products, A, B):
    n = products.shape[0]
    rows = n // _COLS
    p2 = products.reshape(rows, _COLS)
    a2 = A.reshape(rows, _COLS)
    b2 = B.reshape(rows, _COLS)
    grid = (rows + _BLOCK_ROWS - 1) // _BLOCK_ROWS
    spec = pl.BlockSpec((_BLOCK_ROWS, _COLS), lambda i: (i, 0))
    out = pl.pallas_call(
        _body,
        grid=(grid,),
        in_specs=[spec, spec, spec],
        out_specs=spec,
        out_shape=jax.ShapeDtypeStruct((rows, _COLS), jnp.float32),
    )(p2, a2, b2)
    return out.reshape(n)
